# baseline (speedup 1.0000x reference)
import functools
import math

import jax
import jax.numpy as jnp
from jax import lax
from jax.experimental import pallas as pl
from jax.experimental.pallas import tpu as pltpu
from jax.experimental.pallas import tpu_sc as plsc

_NC = 2
_NS = 16
_NW = _NC * _NS

_D = 768
_GROUPS = _D // 16
_K = 32


def _body(ids_hbm, table_hbm, pe_hbm, out_hbm, idx_v, rows_v, pe_v, sem):
    batch, seq_len = ids_hbm.shape
    pos_per_w = seq_len // _NW
    n_chunks = pos_per_w // _K
    scale = jnp.float32(math.sqrt(float(_D)))

    w = lax.axis_index("s") * _NC + lax.axis_index("c")
    w_base = w * pos_per_w

    def row_fma(i, _):
        for j in range(_GROUPS):
            sl = pl.ds(j * 16, 16)
            rows_v[i, sl] = rows_v[i, sl] * scale + pe_v[i, sl]
        return _

    for pc in range(n_chunks):
        base = w_base + pc * _K
        pltpu.sync_copy(pe_hbm.at[pl.ds(base, _K), :], pe_v)
        for b in range(batch):
            pltpu.sync_copy(ids_hbm.at[b, pl.ds(base, _K)], idx_v)
            pltpu.async_copy(table_hbm.at[idx_v], rows_v, sem).wait()
            lax.fori_loop(0, _K, row_fma, 0)
            pltpu.sync_copy(rows_v, out_hbm.at[b, pl.ds(base, _K), :])


def kernel(input_ids, word_embeddings, pe):
    batch, seq_len = input_ids.shape
    ids32 = input_ids.astype(jnp.int32)
    pe2d = pe.reshape(pe.shape[1], pe.shape[2])

    mesh = plsc.VectorSubcoreMesh(
        core_axis_name="c", subcore_axis_name="s",
        num_cores=_NC, num_subcores=_NS,
    )
    run = pl.kernel(
        _body,
        out_type=jax.ShapeDtypeStruct((batch, seq_len, _D), jnp.float32),
        mesh=mesh,
        scratch_types=[
            pltpu.VMEM((_K,), jnp.int32),
            pltpu.VMEM((_K, _D), jnp.float32),
            pltpu.VMEM((_K, _D), jnp.float32),
            pltpu.SemaphoreType.DMA,
        ],
    )
    return run(ids32, word_embeddings, pe2d)
